# Initial kernel scaffold; baseline (speedup 1.0000x reference)
import jax
import jax.numpy as jnp
from jax.experimental import pallas as pl


def kernel(x, W1, b1, W2, b2, We, be):
    raise NotImplementedError("write your pallas kernel here")


# speedup vs baseline: 33.6992x; 33.6992x over previous
import jax
import jax.numpy as jnp
from jax.experimental import pallas as pl
from jax.experimental.pallas import tpu as pltpu

_N, _D, _E = 2048, 1024, 8
_TN = 256


def _leaky(v):
    return jnp.where(v >= 0, v, 0.01 * v)


def _body(x_ref, W1_ref, b1_ref, W2_ref, b2_ref, We_ref, be_ref,
          out_ref, c_ref, acc_ref):
    e = pl.program_id(1)

    @pl.when(e == 0)
    def _router():
        x = x_ref[...]
        h = jnp.dot(x, W1_ref[...], preferred_element_type=jnp.float32)
        h = _leaky(h + b1_ref[...])
        logits = jnp.dot(h, W2_ref[...], preferred_element_type=jnp.float32)
        logits = logits + b2_ref[...]
        eidx = jax.lax.broadcasted_iota(jnp.int32, logits.shape, 1)
        m1 = jnp.max(logits, axis=1, keepdims=True)
        i1 = jnp.min(jnp.where(logits == m1, eidx, _E), axis=1, keepdims=True)
        masked = jnp.where(eidx == i1, -jnp.inf, logits)
        m2 = jnp.max(masked, axis=1, keepdims=True)
        i2 = jnp.min(jnp.where(masked == m2, eidx, _E), axis=1, keepdims=True)
        p2 = 1.0 / (1.0 + jnp.exp(m1 - m2))
        p1 = 1.0 - p2
        c_ref[...] = jnp.where(eidx == i1, p1, 0.0) + jnp.where(eidx == i2, p2, 0.0)

    x = x_ref[...]
    eidx = jax.lax.broadcasted_iota(jnp.int32, c_ref.shape, 1)
    ce = jnp.sum(jnp.where(eidx == e, c_ref[...], 0.0), axis=1, keepdims=True)
    y = jnp.dot(x, We_ref[0], preferred_element_type=jnp.float32) + be_ref[0]
    contrib = ce * y

    @pl.when(e == 0)
    def _init():
        acc_ref[...] = contrib

    @pl.when(e > 0)
    def _acc():
        acc_ref[...] += contrib

    @pl.when(e == _E - 1)
    def _fin():
        out_ref[...] = _leaky(acc_ref[...])


def kernel(x, W1, b1, W2, b2, We, be):
    nt = _N // _TN
    grid = (nt, _E)
    out = pl.pallas_call(
        _body,
        grid=grid,
        in_specs=[
            pl.BlockSpec((_TN, _D), lambda n, e: (n, 0)),
            pl.BlockSpec((_D, _D), lambda n, e: (0, 0)),
            pl.BlockSpec((1, _D), lambda n, e: (0, 0)),
            pl.BlockSpec((_D, _E), lambda n, e: (0, 0)),
            pl.BlockSpec((1, _E), lambda n, e: (0, 0)),
            pl.BlockSpec((1, _D, _D), lambda n, e: (e, 0, 0)),
            pl.BlockSpec((1, 1, _D), lambda n, e: (e, 0, 0)),
        ],
        out_specs=pl.BlockSpec((_TN, _D), lambda n, e: (n, 0)),
        out_shape=jax.ShapeDtypeStruct((_N, _D), jnp.float32),
        scratch_shapes=[
            pltpu.VMEM((_TN, _E), jnp.float32),
            pltpu.VMEM((_TN, _D), jnp.float32),
        ],
        compiler_params=pltpu.CompilerParams(
            dimension_semantics=("arbitrary", "arbitrary"),
        ),
    )(x, W1, b1.reshape(1, _D), W2, b2.reshape(1, _E), We, be.reshape(_E, 1, _D))
    return out


# speedup vs baseline: 48.3250x; 1.4340x over previous
import jax
import jax.numpy as jnp
from jax.experimental import pallas as pl
from jax.experimental.pallas import tpu as pltpu

_N, _D, _E = 2048, 1024, 8
_TN = 1024


def _leaky(v):
    return jnp.where(v >= 0, v, 0.01 * v)


def _body(x_ref, xb_ref, W1_ref, b1_ref, W2_ref, b2_ref, We_ref, be_ref,
          out_ref, c_ref, acc_ref):
    e = pl.program_id(1)

    @pl.when(e == 0)
    def _router():
        x = x_ref[...]
        h = jnp.dot(x, W1_ref[...], preferred_element_type=jnp.float32)
        h = _leaky(h + b1_ref[...])
        logits = jnp.dot(h, W2_ref[...], preferred_element_type=jnp.float32)
        logits = logits + b2_ref[...]
        eidx = jax.lax.broadcasted_iota(jnp.int32, logits.shape, 1)
        m1 = jnp.max(logits, axis=1, keepdims=True)
        i1 = jnp.min(jnp.where(logits == m1, eidx, _E), axis=1, keepdims=True)
        masked = jnp.where(eidx == i1, -jnp.inf, logits)
        m2 = jnp.max(masked, axis=1, keepdims=True)
        i2 = jnp.min(jnp.where(masked == m2, eidx, _E), axis=1, keepdims=True)
        p2 = 1.0 / (1.0 + jnp.exp(m1 - m2))
        p1 = 1.0 - p2
        c_ref[...] = jnp.where(eidx == i1, p1, 0.0) + jnp.where(eidx == i2, p2, 0.0)

    eidx = jax.lax.broadcasted_iota(jnp.int32, c_ref.shape, 1)
    ce = jnp.sum(jnp.where(eidx == e, c_ref[...], 0.0), axis=1, keepdims=True)
    y = jnp.dot(xb_ref[...], We_ref[0], preferred_element_type=jnp.float32)
    contrib = ce * (y + be_ref[0])

    @pl.when(e == 0)
    def _init():
        acc_ref[...] = contrib

    @pl.when(e > 0)
    def _acc():
        acc_ref[...] += contrib

    @pl.when(e == _E - 1)
    def _fin():
        out_ref[...] = _leaky(acc_ref[...])


def kernel(x, W1, b1, W2, b2, We, be):
    nt = _N // _TN
    grid = (nt, _E)
    xb = x.astype(jnp.bfloat16)
    Web = We.astype(jnp.bfloat16)
    out = pl.pallas_call(
        _body,
        grid=grid,
        in_specs=[
            pl.BlockSpec((_TN, _D), lambda n, e: (n, 0)),
            pl.BlockSpec((_TN, _D), lambda n, e: (n, 0)),
            pl.BlockSpec((_D, _D), lambda n, e: (0, 0)),
            pl.BlockSpec((1, _D), lambda n, e: (0, 0)),
            pl.BlockSpec((_D, _E), lambda n, e: (0, 0)),
            pl.BlockSpec((1, _E), lambda n, e: (0, 0)),
            pl.BlockSpec((1, _D, _D), lambda n, e: (e, 0, 0)),
            pl.BlockSpec((1, 1, _D), lambda n, e: (e, 0, 0)),
        ],
        out_specs=pl.BlockSpec((_TN, _D), lambda n, e: (n, 0)),
        out_shape=jax.ShapeDtypeStruct((_N, _D), jnp.float32),
        scratch_shapes=[
            pltpu.VMEM((_TN, _E), jnp.float32),
            pltpu.VMEM((_TN, _D), jnp.float32),
        ],
        compiler_params=pltpu.CompilerParams(
            dimension_semantics=("arbitrary", "arbitrary"),
        ),
    )(x, xb, W1, b1.reshape(1, _D), W2, b2.reshape(1, _E), Web,
      be.reshape(_E, 1, _D))
    return out


# speedup vs baseline: 56.9278x; 1.1780x over previous
import jax
import jax.numpy as jnp
from jax.experimental import pallas as pl
from jax.experimental.pallas import tpu as pltpu

_N, _D, _E = 2048, 1024, 8
_TN = 512


def _leaky(v):
    return jnp.where(v >= 0, v, 0.01 * v)


def _body(x_ref, W1_ref, b1_ref, W2_ref, b2_ref, Wcat_ref, be_ref, out_ref):
    x = x_ref[...]
    h = jnp.dot(x, W1_ref[...], preferred_element_type=jnp.float32)
    h = _leaky(h + b1_ref[...])
    logits = jnp.dot(h, W2_ref[...], preferred_element_type=jnp.float32)
    logits = logits + b2_ref[...]
    eidx = jax.lax.broadcasted_iota(jnp.int32, logits.shape, 1)
    m1 = jnp.max(logits, axis=1, keepdims=True)
    i1 = jnp.min(jnp.where(logits == m1, eidx, _E), axis=1, keepdims=True)
    masked = jnp.where(eidx == i1, -jnp.inf, logits)
    m2 = jnp.max(masked, axis=1, keepdims=True)
    i2 = jnp.min(jnp.where(masked == m2, eidx, _E), axis=1, keepdims=True)
    p2 = 1.0 / (1.0 + jnp.exp(m1 - m2))
    p1 = 1.0 - p2
    c = jnp.where(eidx == i1, p1, 0.0) + jnp.where(eidx == i2, p2, 0.0)

    xcat = jnp.concatenate(
        [(c[:, e:e + 1] * x).astype(jnp.bfloat16) for e in range(_E)], axis=1)
    y = jnp.dot(xcat, Wcat_ref[...], preferred_element_type=jnp.float32)
    bias = jnp.dot(c, be_ref[...], preferred_element_type=jnp.float32)
    out_ref[...] = _leaky(y + bias)


def kernel(x, W1, b1, W2, b2, We, be):
    nt = _N // _TN
    Wcat = We.astype(jnp.bfloat16).reshape(_E * _D, _D)
    out = pl.pallas_call(
        _body,
        grid=(nt,),
        in_specs=[
            pl.BlockSpec((_TN, _D), lambda n: (n, 0)),
            pl.BlockSpec((_D, _D), lambda n: (0, 0)),
            pl.BlockSpec((1, _D), lambda n: (0, 0)),
            pl.BlockSpec((_D, _E), lambda n: (0, 0)),
            pl.BlockSpec((1, _E), lambda n: (0, 0)),
            pl.BlockSpec((_E * _D, _D), lambda n: (0, 0)),
            pl.BlockSpec((_E, _D), lambda n: (0, 0)),
        ],
        out_specs=pl.BlockSpec((_TN, _D), lambda n: (n, 0)),
        out_shape=jax.ShapeDtypeStruct((_N, _D), jnp.float32),
        compiler_params=pltpu.CompilerParams(
            dimension_semantics=("arbitrary",),
        ),
    )(x, W1, b1.reshape(1, _D), W2, b2.reshape(1, _E), Wcat, be)
    return out


# speedup vs baseline: 57.3361x; 1.0072x over previous
import jax
import jax.numpy as jnp
from jax.experimental import pallas as pl
from jax.experimental.pallas import tpu as pltpu

_N, _D, _E = 2048, 1024, 8
_TN = 1024


def _leaky(v):
    return jnp.where(v >= 0, v, 0.01 * v)


def _body(x_ref, W1_ref, b1_ref, W2_ref, b2_ref, Wcat_ref, be_ref, out_ref):
    x = x_ref[...]
    h = jnp.dot(x, W1_ref[...], preferred_element_type=jnp.float32)
    h = _leaky(h + b1_ref[...])
    logits = jnp.dot(h, W2_ref[...], preferred_element_type=jnp.float32)
    logits = logits + b2_ref[...]
    eidx = jax.lax.broadcasted_iota(jnp.int32, logits.shape, 1)
    m1 = jnp.max(logits, axis=1, keepdims=True)
    i1 = jnp.min(jnp.where(logits == m1, eidx, _E), axis=1, keepdims=True)
    masked = jnp.where(eidx == i1, -jnp.inf, logits)
    m2 = jnp.max(masked, axis=1, keepdims=True)
    i2 = jnp.min(jnp.where(masked == m2, eidx, _E), axis=1, keepdims=True)
    p2 = 1.0 / (1.0 + jnp.exp(m1 - m2))
    p1 = 1.0 - p2
    c = jnp.where(eidx == i1, p1, 0.0) + jnp.where(eidx == i2, p2, 0.0)

    xcat = jnp.concatenate(
        [(c[:, e:e + 1] * x).astype(jnp.bfloat16) for e in range(_E)], axis=1)
    y = jnp.dot(xcat, Wcat_ref[...], preferred_element_type=jnp.float32)
    bias = jnp.dot(c, be_ref[...], preferred_element_type=jnp.float32)
    out_ref[...] = _leaky(y + bias)


def kernel(x, W1, b1, W2, b2, We, be):
    nt = _N // _TN
    Wcat = We.astype(jnp.bfloat16).reshape(_E * _D, _D)
    out = pl.pallas_call(
        _body,
        grid=(nt,),
        in_specs=[
            pl.BlockSpec((_TN, _D), lambda n: (n, 0)),
            pl.BlockSpec((_D, _D), lambda n: (0, 0)),
            pl.BlockSpec((1, _D), lambda n: (0, 0)),
            pl.BlockSpec((_D, _E), lambda n: (0, 0)),
            pl.BlockSpec((1, _E), lambda n: (0, 0)),
            pl.BlockSpec((_E * _D, _D), lambda n: (0, 0)),
            pl.BlockSpec((_E, _D), lambda n: (0, 0)),
        ],
        out_specs=pl.BlockSpec((_TN, _D), lambda n: (n, 0)),
        out_shape=jax.ShapeDtypeStruct((_N, _D), jnp.float32),
        compiler_params=pltpu.CompilerParams(
            dimension_semantics=("arbitrary",),
        ),
    )(x, W1, b1.reshape(1, _D), W2, b2.reshape(1, _E), Wcat, be)
    return out
